# baseline (speedup 1.0000x reference)
import functools

import jax
import jax.numpy as jnp
from jax import lax
from jax.experimental import pallas as pl
from jax.experimental.pallas import tpu as pltpu
from jax.experimental.pallas import tpu_sc as plsc

B, L, V, D, T, TW, F = 1024, 20, 100000, 64, 26, 2, 10
NC, NS = 2, 16
NW = NC * NS
NB = B // NW
NBL = NB * L
NCH = NBL // 128
LANE = 16
VP = 102400
BV = 2048
TG = 13
NGRP = NB // LANE


def _contract_body(w_ref, tab_ref, out_ref):
    t3 = tab_ref[...]
    w3 = w_ref[0]
    out_ref[...] = jnp.sum(t3 * w3[:, :, None], axis=1)


def _contract(tabT, w3, grp, nt):
    last = (V - 1) // BV
    return pl.pallas_call(
        _contract_body,
        grid=(VP // BV,),
        in_specs=[
            pl.BlockSpec((1, nt, D), lambda vc: (grp, 0, 0)),
            pl.BlockSpec((nt, D, BV),
                         lambda vc: (grp, 0, jnp.minimum(vc, last))),
        ],
        out_specs=pl.BlockSpec((nt, BV), lambda vc: (0, vc)),
        out_shape=jax.ShapeDtypeStruct((nt, VP), jnp.float32),
        compiler_params=pltpu.CompilerParams(
            dimension_semantics=("parallel",)),
    )(w3, tabT)


def _wid_base():
    c = lax.axis_index("c")
    s = lax.axis_index("s")
    wid = s * NC + c
    return wid, wid * NB


def _lanes():
    return lax.iota(jnp.int32, LANE)


def _hsum(v):
    lanes = _lanes()
    dnums = lax.GatherDimensionNumbers(
        offset_dims=(), collapsed_slice_dims=(0,), start_index_map=(0,))
    for sh in (8, 4, 2, 1):
        perm = jnp.bitwise_xor(lanes, sh)
        v = v + lax.gather(
            v, perm[:, None], dnums, slice_sizes=(1,),
            mode=lax.GatherScatterMode.PROMISE_IN_BOUNDS)
    return v


def _issue_table(src, idxall, scals, sem, t, row0=0):
    for j in range(NCH):
        r = row0 + t * NCH + j
        pltpu.async_copy(src.at[idxall.at[r]], scals.at[r], sem)


def _drain_table(dummy, dummyv, sem):
    pltpu.make_async_copy(dummy, dummyv, sem).wait()


def _pool_step(scals, accs, t, row0, wts1=None, woff=0):
    lanes = _lanes()
    base20 = lanes * L
    out = []
    for g in range(NGRP):
        a = accs[g]
        for l in range(L):
            pv = base20 + (g * LANE * L + l)
            row = (row0 + t * NCH) + lax.shift_right_logical(pv, 7)
            col = jnp.bitwise_and(pv, 127)
            vals = plsc.load_gather(scals, [row, col])
            if wts1 is not None:
                wl = plsc.load_gather(wts1, [woff + pv])
                a = a + vals * wl
            else:
                a = a + vals
        out.append(a)
    return tuple(out)


def _dense_part(ffv, wvecv):
    lanes = _lanes()
    wv = wvecv[:]
    dense = []
    for g in range(NGRP):
        def dense_body(k, dv):
            x = _hsum(ffv[g * LANE + k, :] * wv)
            return jnp.where(lanes == k, dv + x, dv)

        dense.append(lax.fori_loop(0, LANE, dense_body,
                                   jnp.zeros((LANE,), jnp.float32)))
    return dense


def _sc_pool_body(contrib, gidxw, ffp, wvec, dummy, acc_out,
                  idxall, scals, ffv, wvecv, outv, dummyv, sem):
    wid, base = _wid_base()
    pltpu.sync_copy(gidxw.at[wid], idxall)
    pltpu.sync_copy(ffp.at[pl.ds(base, NB)], ffv)
    pltpu.sync_copy(wvec, wvecv)

    _issue_table(contrib, idxall, scals, sem, 0)
    dense = _dense_part(ffv, wvecv)

    def table_body(t, carry):
        @pl.when(t + 1 < TG)
        def _():
            _issue_table(contrib, idxall, scals, sem, t + 1)
        _drain_table(dummy, dummyv, sem)
        return _pool_step(scals, carry, t, 0)

    accs = lax.fori_loop(0, TG, table_body,
                         (dense[0], dense[1]))
    for g in range(NGRP):
        outv[pl.ds(g * LANE, LANE)] = accs[g]
    pltpu.sync_copy(outv, acc_out.at[pl.ds(base, NB)])


_WROW0 = TG * NCH


def _sc_final_body(acc_in, contrib, contrib2, gidxw, gidx2w, wtsr, out,
                   idxall, scals, wtsv, accv, outv, dummyv, sem):
    wid, base = _wid_base()
    pltpu.sync_copy(gidxw.at[wid], idxall.at[pl.ds(0, TG * NCH)])
    pltpu.sync_copy(gidx2w.at[wid], idxall.at[pl.ds(_WROW0, TW * NCH)])
    pltpu.sync_copy(wtsr.at[wid], wtsv)
    pltpu.sync_copy(acc_in.at[pl.ds(base, NB)], accv)

    _issue_table(contrib, idxall, scals, sem, 0)

    def table_body(t, carry):
        @pl.when(t + 1 < TG)
        def _():
            _issue_table(contrib, idxall, scals, sem, t + 1)
        _drain_table(contrib2, dummyv, sem)
        return _pool_step(scals, carry, t, 0)

    zero = jnp.zeros((LANE,), jnp.float32)
    accs = lax.fori_loop(0, TG, table_body, (zero, zero))

    _issue_table(contrib2, idxall, scals, sem, 0, row0=_WROW0)
    _issue_table(contrib2, idxall, scals, sem, 1, row0=_WROW0)
    for tw in range(TW):
        _drain_table(contrib2, dummyv, sem)
        accs = _pool_step(scals, accs, tw, _WROW0,
                          wts1=wtsv, woff=tw * NBL)

    for g in range(NGRP):
        x = accs[g] + accv[pl.ds(g * LANE, LANE)]
        outv[pl.ds(g * LANE, LANE)] = 1.0 / (1.0 + jnp.exp(-x))
    pltpu.sync_copy(outv, out.at[pl.ds(base, NB)])


def _sc_params():
    return dict(
        mesh=plsc.VectorSubcoreMesh(core_axis_name="c", subcore_axis_name="s"),
        compiler_params=pltpu.CompilerParams(use_tc_tiling_on_sc=False,
                                             needs_layout_passes=False),
    )


@functools.cache
def _sc_pool_kernel():
    return functools.partial(
        pl.kernel,
        out_type=jax.ShapeDtypeStruct((B,), jnp.float32),
        scratch_types=[
            pltpu.VMEM((TG * NCH, 128), jnp.int32),
            pltpu.VMEM((TG * NCH, 128), jnp.float32),
            pltpu.VMEM((NB, LANE), jnp.float32),
            pltpu.VMEM((LANE,), jnp.float32),
            pltpu.VMEM((NB,), jnp.float32),
            pltpu.VMEM((NCH, 128), jnp.float32),
            pltpu.SemaphoreType.DMA,
        ],
        **_sc_params(),
    )(_sc_pool_body)


@functools.cache
def _sc_final_kernel():
    nrow = (TG + TW) * NCH
    return functools.partial(
        pl.kernel,
        out_type=jax.ShapeDtypeStruct((B,), jnp.float32),
        scratch_types=[
            pltpu.VMEM((nrow, 128), jnp.int32),
            pltpu.VMEM((nrow, 128), jnp.float32),
            pltpu.VMEM((TW * NBL,), jnp.float32),
            pltpu.VMEM((NB,), jnp.float32),
            pltpu.VMEM((NB,), jnp.float32),
            pltpu.VMEM((NCH, 128), jnp.float32),
            pltpu.SemaphoreType.DMA,
        ],
        **_sc_params(),
    )(_sc_final_body)


def _group_idx(idx_slab, nt):
    offs = (jnp.arange(nt, dtype=jnp.int32) * VP)[:, None, None]
    g = (idx_slab.astype(jnp.int32) + offs).reshape(nt, NW, NCH, 128)
    return jnp.transpose(g, (1, 0, 2, 3)).reshape(NW, nt * NCH, 128)


def kernel(float_features, tables, weighted_tables, idscore_weights,
           Wd, bd, Wo, bo, idlist_indices, idscore_indices):
    wo_mean = jnp.mean(Wo, axis=1)
    wd_vec = Wd @ wo_mean[:8]
    const = jnp.dot(bd, wo_mean[:8]) + jnp.mean(bo)
    wvec = jnp.concatenate(
        [wd_vec, const[None], jnp.zeros((LANE - F - 1,), jnp.float32)])
    ffp = jnp.concatenate(
        [float_features,
         jnp.ones((B, 1), jnp.float32),
         jnp.zeros((B, LANE - F - 1), jnp.float32)], axis=1)
    wot = wo_mean[8:8 + T * D].reshape(T, D)
    wow = wo_mean[8 + T * D:].reshape(TW, D)

    tabT = jnp.transpose(tables, (0, 2, 1))
    wtabT = jnp.transpose(weighted_tables, (0, 2, 1))
    wot3 = wot.reshape(2, TG, D)
    wow3 = wow.reshape(1, TW, D)
    contribA = _contract(tabT, wot3, 0, TG).reshape(TG * VP)
    contribB = _contract(tabT, wot3, 1, TG).reshape(TG * VP)
    contrib2 = _contract(wtabT, wow3, 0, TW).reshape(TW * VP)

    gidxA = _group_idx(idlist_indices[:TG], TG)
    gidxB = _group_idx(idlist_indices[TG:], TG)
    gidxW = _group_idx(idscore_indices, TW)
    wtsr = jnp.transpose(idscore_weights.reshape(TW, NW, NBL),
                         (1, 0, 2)).reshape(NW, TW * NBL)
    dummy = jnp.zeros((NCH, 128), jnp.float32)

    acc = _sc_pool_kernel()(contribA, gidxA, ffp, wvec, dummy)
    return _sc_final_kernel()(acc, contribB, contrib2, gidxB, gidxW, wtsr)


# speedup vs baseline: 1.0456x; 1.0456x over previous
import functools

import jax
import jax.numpy as jnp
from jax import lax
from jax.experimental import pallas as pl
from jax.experimental.pallas import tpu as pltpu
from jax.experimental.pallas import tpu_sc as plsc

B, L, V, D, T, TW, F = 1024, 20, 100000, 64, 26, 2, 10
NC, NS = 2, 16
NW = NC * NS
NB = B // NW
NBL = NB * L
NCH = NBL // 128
LANE = 16
VP = 102400
BV = 4096
TG = 13
NGRP = NB // LANE


def _contract_body(w_ref, tab_ref, out_ref):
    t3 = tab_ref[...]
    w3 = w_ref[0]
    out_ref[...] = jnp.sum(t3 * w3[:, :, None], axis=1)


def _contract(tabT, w3, grp, nt):
    last = (V - 1) // BV
    return pl.pallas_call(
        _contract_body,
        grid=(VP // BV,),
        in_specs=[
            pl.BlockSpec((1, nt, D), lambda vc: (grp, 0, 0)),
            pl.BlockSpec((nt, D, BV),
                         lambda vc: (grp, 0, jnp.minimum(vc, last))),
        ],
        out_specs=pl.BlockSpec((nt, BV), lambda vc: (0, vc)),
        out_shape=jax.ShapeDtypeStruct((nt, VP), jnp.float32),
        compiler_params=pltpu.CompilerParams(
            dimension_semantics=("parallel",)),
    )(w3, tabT)


def _wid_base():
    c = lax.axis_index("c")
    s = lax.axis_index("s")
    wid = s * NC + c
    return wid, wid * NB


def _lanes():
    return lax.iota(jnp.int32, LANE)


def _hsum(v):
    lanes = _lanes()
    dnums = lax.GatherDimensionNumbers(
        offset_dims=(), collapsed_slice_dims=(0,), start_index_map=(0,))
    for sh in (8, 4, 2, 1):
        perm = jnp.bitwise_xor(lanes, sh)
        v = v + lax.gather(
            v, perm[:, None], dnums, slice_sizes=(1,),
            mode=lax.GatherScatterMode.PROMISE_IN_BOUNDS)
    return v


def _issue_table(src, idxall, scals, sem, t, row0=0):
    for j in range(NCH):
        r = row0 + t * NCH + j
        pltpu.async_copy(src.at[idxall.at[r]], scals.at[r], sem)


def _drain_table(dummy, dummyv, sem):
    pltpu.make_async_copy(dummy, dummyv, sem).wait()


def _pool_step(scals, accs, t, row0, wts1=None, woff=0):
    lanes = _lanes()
    base20 = lanes * L
    out = []
    for g in range(NGRP):
        a = accs[g]
        for l in range(L):
            pv = base20 + (g * LANE * L + l)
            row = (row0 + t * NCH) + lax.shift_right_logical(pv, 7)
            col = jnp.bitwise_and(pv, 127)
            vals = plsc.load_gather(scals, [row, col])
            if wts1 is not None:
                wl = plsc.load_gather(wts1, [woff + pv])
                a = a + vals * wl
            else:
                a = a + vals
        out.append(a)
    return tuple(out)


def _dense_part(ffv, wvecv):
    lanes = _lanes()
    wv = wvecv[:]
    dense = []
    for g in range(NGRP):
        def dense_body(k, dv):
            x = _hsum(ffv[g * LANE + k, :] * wv)
            return jnp.where(lanes == k, dv + x, dv)

        dense.append(lax.fori_loop(0, LANE, dense_body,
                                   jnp.zeros((LANE,), jnp.float32)))
    return dense


def _sc_pool_body(contrib, gidxw, ffp, wvec, dummy, acc_out,
                  idxall, scals, ffv, wvecv, outv, dummyv, sem):
    wid, base = _wid_base()
    pltpu.sync_copy(gidxw.at[wid], idxall)
    pltpu.sync_copy(ffp.at[pl.ds(base, NB)], ffv)
    pltpu.sync_copy(wvec, wvecv)

    _issue_table(contrib, idxall, scals, sem, 0)
    dense = _dense_part(ffv, wvecv)

    def table_body(t, carry):
        @pl.when(t + 1 < TG)
        def _():
            _issue_table(contrib, idxall, scals, sem, t + 1)
        _drain_table(dummy, dummyv, sem)
        return _pool_step(scals, carry, t, 0)

    accs = lax.fori_loop(0, TG, table_body,
                         (dense[0], dense[1]))
    for g in range(NGRP):
        outv[pl.ds(g * LANE, LANE)] = accs[g]
    pltpu.sync_copy(outv, acc_out.at[pl.ds(base, NB)])


_WROW0 = TG * NCH


def _sc_final_body(acc_in, contrib, contrib2, gidxw, gidx2w, wtsr, out,
                   idxall, scals, wtsv, accv, outv, dummyv, sem):
    wid, base = _wid_base()
    pltpu.sync_copy(gidxw.at[wid], idxall.at[pl.ds(0, TG * NCH)])
    pltpu.sync_copy(gidx2w.at[wid], idxall.at[pl.ds(_WROW0, TW * NCH)])
    pltpu.sync_copy(wtsr.at[wid], wtsv)
    pltpu.sync_copy(acc_in.at[pl.ds(base, NB)], accv)

    _issue_table(contrib, idxall, scals, sem, 0)

    def table_body(t, carry):
        @pl.when(t + 1 < TG)
        def _():
            _issue_table(contrib, idxall, scals, sem, t + 1)
        _drain_table(contrib2, dummyv, sem)
        return _pool_step(scals, carry, t, 0)

    zero = jnp.zeros((LANE,), jnp.float32)
    accs = lax.fori_loop(0, TG, table_body, (zero, zero))

    _issue_table(contrib2, idxall, scals, sem, 0, row0=_WROW0)
    _issue_table(contrib2, idxall, scals, sem, 1, row0=_WROW0)
    for tw in range(TW):
        _drain_table(contrib2, dummyv, sem)
        accs = _pool_step(scals, accs, tw, _WROW0,
                          wts1=wtsv, woff=tw * NBL)

    for g in range(NGRP):
        x = accs[g] + accv[pl.ds(g * LANE, LANE)]
        outv[pl.ds(g * LANE, LANE)] = 1.0 / (1.0 + jnp.exp(-x))
    pltpu.sync_copy(outv, out.at[pl.ds(base, NB)])


def _sc_params():
    return dict(
        mesh=plsc.VectorSubcoreMesh(core_axis_name="c", subcore_axis_name="s"),
        compiler_params=pltpu.CompilerParams(use_tc_tiling_on_sc=False,
                                             needs_layout_passes=False),
    )


@functools.cache
def _sc_pool_kernel():
    return functools.partial(
        pl.kernel,
        out_type=jax.ShapeDtypeStruct((B,), jnp.float32),
        scratch_types=[
            pltpu.VMEM((TG * NCH, 128), jnp.int32),
            pltpu.VMEM((TG * NCH, 128), jnp.float32),
            pltpu.VMEM((NB, LANE), jnp.float32),
            pltpu.VMEM((LANE,), jnp.float32),
            pltpu.VMEM((NB,), jnp.float32),
            pltpu.VMEM((NCH, 128), jnp.float32),
            pltpu.SemaphoreType.DMA,
        ],
        **_sc_params(),
    )(_sc_pool_body)


@functools.cache
def _sc_final_kernel():
    nrow = (TG + TW) * NCH
    return functools.partial(
        pl.kernel,
        out_type=jax.ShapeDtypeStruct((B,), jnp.float32),
        scratch_types=[
            pltpu.VMEM((nrow, 128), jnp.int32),
            pltpu.VMEM((nrow, 128), jnp.float32),
            pltpu.VMEM((TW * NBL,), jnp.float32),
            pltpu.VMEM((NB,), jnp.float32),
            pltpu.VMEM((NB,), jnp.float32),
            pltpu.VMEM((NCH, 128), jnp.float32),
            pltpu.SemaphoreType.DMA,
        ],
        **_sc_params(),
    )(_sc_final_body)


def _group_idx(idx_slab, nt):
    offs = (jnp.arange(nt, dtype=jnp.int32) * VP)[:, None, None]
    g = (idx_slab.astype(jnp.int32) + offs).reshape(nt, NW, NCH, 128)
    return jnp.transpose(g, (1, 0, 2, 3)).reshape(NW, nt * NCH, 128)


def kernel(float_features, tables, weighted_tables, idscore_weights,
           Wd, bd, Wo, bo, idlist_indices, idscore_indices):
    wo_mean = jnp.mean(Wo, axis=1)
    wd_vec = Wd @ wo_mean[:8]
    const = jnp.dot(bd, wo_mean[:8]) + jnp.mean(bo)
    wvec = jnp.concatenate(
        [wd_vec, const[None], jnp.zeros((LANE - F - 1,), jnp.float32)])
    ffp = jnp.concatenate(
        [float_features,
         jnp.ones((B, 1), jnp.float32),
         jnp.zeros((B, LANE - F - 1), jnp.float32)], axis=1)
    wot = wo_mean[8:8 + T * D].reshape(T, D)
    wow = wo_mean[8 + T * D:].reshape(TW, D)

    tabT = jnp.transpose(tables, (0, 2, 1))
    wtabT = jnp.transpose(weighted_tables, (0, 2, 1))
    wot3 = wot.reshape(2, TG, D)
    wow3 = wow.reshape(1, TW, D)
    contribA = _contract(tabT, wot3, 0, TG).reshape(TG * VP)
    contribB = _contract(tabT, wot3, 1, TG).reshape(TG * VP)
    contrib2 = _contract(wtabT, wow3, 0, TW).reshape(TW * VP)

    gidxA = _group_idx(idlist_indices[:TG], TG)
    gidxB = _group_idx(idlist_indices[TG:], TG)
    gidxW = _group_idx(idscore_indices, TW)
    wtsr = jnp.transpose(idscore_weights.reshape(TW, NW, NBL),
                         (1, 0, 2)).reshape(NW, TW * NBL)
    dummy = jnp.zeros((NCH, 128), jnp.float32)

    acc = _sc_pool_kernel()(contribA, gidxA, ffp, wvec, dummy)
    return _sc_final_kernel()(acc, contribB, contrib2, gidxB, gidxW, wtsr)


# speedup vs baseline: 1.1096x; 1.0612x over previous
import functools

import jax
import jax.numpy as jnp
from jax import lax
from jax.experimental import pallas as pl
from jax.experimental.pallas import tpu as pltpu
from jax.experimental.pallas import tpu_sc as plsc

B, L, V, D, T, TW, F = 1024, 20, 100000, 64, 26, 2, 10
NC, NS = 2, 16
NW = NC * NS
NB = B // NW
NBL = NB * L
NCH = NBL // 128
LANE = 16
VP = 102400
BV = 51200
TG = 13
NGRP = NB // LANE


def _contract_body(w_ref, tab_ref, out_ref):
    t2 = tab_ref[0]
    w1 = w_ref[0, 0]
    out_ref[0, :] = jnp.sum(t2 * w1[:, None], axis=0)


def _contract(tabT, wr, grp, nt):
    last = (V - 1) // BV
    return pl.pallas_call(
        _contract_body,
        grid=(nt, VP // BV),
        in_specs=[
            pl.BlockSpec((1, 1, D), lambda t, vc: (grp * nt + t, 0, 0)),
            pl.BlockSpec((1, D, BV),
                         lambda t, vc: (grp * nt + t, 0,
                                        jnp.minimum(vc, last))),
        ],
        out_specs=pl.BlockSpec(
            (1, BV), lambda t, vc: (0, t * (VP // BV) + vc)),
        out_shape=jax.ShapeDtypeStruct((1, nt * VP), jnp.float32),
    )(wr, tabT)


def _wid_base():
    c = lax.axis_index("c")
    s = lax.axis_index("s")
    wid = s * NC + c
    return wid, wid * NB


def _lanes():
    return lax.iota(jnp.int32, LANE)


def _hsum(v):
    lanes = _lanes()
    dnums = lax.GatherDimensionNumbers(
        offset_dims=(), collapsed_slice_dims=(0,), start_index_map=(0,))
    for sh in (8, 4, 2, 1):
        perm = jnp.bitwise_xor(lanes, sh)
        v = v + lax.gather(
            v, perm[:, None], dnums, slice_sizes=(1,),
            mode=lax.GatherScatterMode.PROMISE_IN_BOUNDS)
    return v


def _issue_table(src, idxall, scals, sem, t, row0=0):
    for j in range(NCH):
        r = row0 + t * NCH + j
        pltpu.async_copy(src.at[idxall.at[r]], scals.at[r], sem)


def _drain_table(dummy, dummyv, sem):
    pltpu.make_async_copy(dummy, dummyv, sem).wait()


def _pool_step(scals, accs, t, row0, wts1=None, woff=0):
    lanes = _lanes()
    base20 = lanes * L
    out = []
    for g in range(NGRP):
        a = accs[g]
        for l in range(L):
            pv = base20 + (g * LANE * L + l)
            row = (row0 + t * NCH) + lax.shift_right_logical(pv, 7)
            col = jnp.bitwise_and(pv, 127)
            vals = plsc.load_gather(scals, [row, col])
            if wts1 is not None:
                wl = plsc.load_gather(wts1, [woff + pv])
                a = a + vals * wl
            else:
                a = a + vals
        out.append(a)
    return tuple(out)


def _dense_part(ffv, wvecv):
    lanes = _lanes()
    wv = wvecv[:]
    dense = []
    for g in range(NGRP):
        def dense_body(k, dv):
            x = _hsum(ffv[g * LANE + k, :] * wv)
            return jnp.where(lanes == k, dv + x, dv)

        dense.append(lax.fori_loop(0, LANE, dense_body,
                                   jnp.zeros((LANE,), jnp.float32)))
    return dense


def _sc_pool_body(contrib, gidxw, ffp, wvec, dummy, acc_out,
                  idxall, scals, ffv, wvecv, outv, dummyv, sem):
    wid, base = _wid_base()
    pltpu.sync_copy(gidxw.at[wid], idxall)
    pltpu.sync_copy(ffp.at[pl.ds(base, NB)], ffv)
    pltpu.sync_copy(wvec, wvecv)

    _issue_table(contrib, idxall, scals, sem, 0)
    dense = _dense_part(ffv, wvecv)

    def table_body(t, carry):
        @pl.when(t + 1 < TG)
        def _():
            _issue_table(contrib, idxall, scals, sem, t + 1)
        _drain_table(dummy, dummyv, sem)
        return _pool_step(scals, carry, t, 0)

    accs = lax.fori_loop(0, TG, table_body,
                         (dense[0], dense[1]))
    for g in range(NGRP):
        outv[pl.ds(g * LANE, LANE)] = accs[g]
    pltpu.sync_copy(outv, acc_out.at[pl.ds(base, NB)])


_WROW0 = TG * NCH


def _sc_final_body(acc_in, contrib, contrib2, gidxw, gidx2w, wtsr, out,
                   idxall, scals, wtsv, accv, outv, dummyv, sem):
    wid, base = _wid_base()
    pltpu.sync_copy(gidxw.at[wid], idxall.at[pl.ds(0, TG * NCH)])
    pltpu.sync_copy(gidx2w.at[wid], idxall.at[pl.ds(_WROW0, TW * NCH)])
    pltpu.sync_copy(wtsr.at[wid], wtsv)
    pltpu.sync_copy(acc_in.at[pl.ds(base, NB)], accv)

    _issue_table(contrib, idxall, scals, sem, 0)

    def table_body(t, carry):
        @pl.when(t + 1 < TG)
        def _():
            _issue_table(contrib, idxall, scals, sem, t + 1)
        _drain_table(contrib2, dummyv, sem)
        return _pool_step(scals, carry, t, 0)

    zero = jnp.zeros((LANE,), jnp.float32)
    accs = lax.fori_loop(0, TG, table_body, (zero, zero))

    _issue_table(contrib2, idxall, scals, sem, 0, row0=_WROW0)
    _issue_table(contrib2, idxall, scals, sem, 1, row0=_WROW0)
    for tw in range(TW):
        _drain_table(contrib2, dummyv, sem)
        accs = _pool_step(scals, accs, tw, _WROW0,
                          wts1=wtsv, woff=tw * NBL)

    for g in range(NGRP):
        x = accs[g] + accv[pl.ds(g * LANE, LANE)]
        outv[pl.ds(g * LANE, LANE)] = 1.0 / (1.0 + jnp.exp(-x))
    pltpu.sync_copy(outv, out.at[pl.ds(base, NB)])


def _sc_params():
    return dict(
        mesh=plsc.VectorSubcoreMesh(core_axis_name="c", subcore_axis_name="s"),
        compiler_params=pltpu.CompilerParams(use_tc_tiling_on_sc=False,
                                             needs_layout_passes=False),
    )


@functools.cache
def _sc_pool_kernel():
    return functools.partial(
        pl.kernel,
        out_type=jax.ShapeDtypeStruct((B,), jnp.float32),
        scratch_types=[
            pltpu.VMEM((TG * NCH, 128), jnp.int32),
            pltpu.VMEM((TG * NCH, 128), jnp.float32),
            pltpu.VMEM((NB, LANE), jnp.float32),
            pltpu.VMEM((LANE,), jnp.float32),
            pltpu.VMEM((NB,), jnp.float32),
            pltpu.VMEM((NCH, 128), jnp.float32),
            pltpu.SemaphoreType.DMA,
        ],
        **_sc_params(),
    )(_sc_pool_body)


@functools.cache
def _sc_final_kernel():
    nrow = (TG + TW) * NCH
    return functools.partial(
        pl.kernel,
        out_type=jax.ShapeDtypeStruct((B,), jnp.float32),
        scratch_types=[
            pltpu.VMEM((nrow, 128), jnp.int32),
            pltpu.VMEM((nrow, 128), jnp.float32),
            pltpu.VMEM((TW * NBL,), jnp.float32),
            pltpu.VMEM((NB,), jnp.float32),
            pltpu.VMEM((NB,), jnp.float32),
            pltpu.VMEM((NCH, 128), jnp.float32),
            pltpu.SemaphoreType.DMA,
        ],
        **_sc_params(),
    )(_sc_final_body)


def _group_idx(idx_slab, nt):
    offs = (jnp.arange(nt, dtype=jnp.int32) * VP)[:, None, None]
    g = (idx_slab.astype(jnp.int32) + offs).reshape(nt, NW, NCH, 128)
    return jnp.transpose(g, (1, 0, 2, 3)).reshape(NW, nt * NCH, 128)


def kernel(float_features, tables, weighted_tables, idscore_weights,
           Wd, bd, Wo, bo, idlist_indices, idscore_indices):
    wo_mean = jnp.mean(Wo, axis=1)
    wd_vec = Wd @ wo_mean[:8]
    const = jnp.dot(bd, wo_mean[:8]) + jnp.mean(bo)
    wvec = jnp.concatenate(
        [wd_vec, const[None], jnp.zeros((LANE - F - 1,), jnp.float32)])
    ffp = jnp.concatenate(
        [float_features,
         jnp.ones((B, 1), jnp.float32),
         jnp.zeros((B, LANE - F - 1), jnp.float32)], axis=1)
    wot = wo_mean[8:8 + T * D].reshape(T, D)
    wow = wo_mean[8 + T * D:].reshape(TW, D)

    tabT = jnp.transpose(tables, (0, 2, 1))
    wtabT = jnp.transpose(weighted_tables, (0, 2, 1))
    wot3 = wot.reshape(T, 1, D)
    wow3 = wow.reshape(TW, 1, D)
    contribA = _contract(tabT, wot3, 0, TG).reshape(TG * VP)
    contribB = _contract(tabT, wot3, 1, TG).reshape(TG * VP)
    contrib2 = _contract(wtabT, wow3, 0, TW).reshape(TW * VP)

    gidxA = _group_idx(idlist_indices[:TG], TG)
    gidxB = _group_idx(idlist_indices[TG:], TG)
    gidxW = _group_idx(idscore_indices, TW)
    wtsr = jnp.transpose(idscore_weights.reshape(TW, NW, NBL),
                         (1, 0, 2)).reshape(NW, TW * NBL)
    dummy = jnp.zeros((NCH, 128), jnp.float32)

    acc = _sc_pool_kernel()(contribA, gidxA, ffp, wvec, dummy)
    return _sc_final_kernel()(acc, contribB, contrib2, gidxB, gidxW, wtsr)


# speedup vs baseline: 1.1156x; 1.0054x over previous
import functools

import jax
import jax.numpy as jnp
from jax import lax
from jax.experimental import pallas as pl
from jax.experimental.pallas import tpu as pltpu
from jax.experimental.pallas import tpu_sc as plsc

B, L, V, D, T, TW, F = 1024, 20, 100000, 64, 26, 2, 10
NC, NS = 2, 16
NW = NC * NS
NB = B // NW
NBL = NB * L
NCH = NBL // 128
LANE = 16
VP = 102400
BV = 51200
TG = 13
NGRP = NB // LANE


BH = BV // 2


def _contract_body(w_ref, tlo_ref, thi_ref, out_ref):
    w1 = w_ref[0, 0]
    out_ref[0, :BH] = jnp.sum(tlo_ref[0] * w1[:, None], axis=0)
    out_ref[0, BH:] = jnp.sum(thi_ref[0] * w1[:, None], axis=0)


def _contract(tabT, wr, grp, nt):
    lasth = (V - 1) // BH

    def tab_spec(half):
        return pl.BlockSpec(
            (1, D, BH),
            lambda t, vc: (grp * nt + t, 0,
                           jnp.minimum(2 * vc + half, lasth)))

    return pl.pallas_call(
        _contract_body,
        grid=(nt, VP // BV),
        in_specs=[
            pl.BlockSpec((1, 1, D), lambda t, vc: (grp * nt + t, 0, 0)),
            tab_spec(0),
            tab_spec(1),
        ],
        out_specs=pl.BlockSpec(
            (1, BV), lambda t, vc: (0, t * (VP // BV) + vc)),
        out_shape=jax.ShapeDtypeStruct((1, nt * VP), jnp.float32),
    )(wr, tabT, tabT)


def _wid_base():
    c = lax.axis_index("c")
    s = lax.axis_index("s")
    wid = s * NC + c
    return wid, wid * NB


def _lanes():
    return lax.iota(jnp.int32, LANE)


def _hsum(v):
    lanes = _lanes()
    dnums = lax.GatherDimensionNumbers(
        offset_dims=(), collapsed_slice_dims=(0,), start_index_map=(0,))
    for sh in (8, 4, 2, 1):
        perm = jnp.bitwise_xor(lanes, sh)
        v = v + lax.gather(
            v, perm[:, None], dnums, slice_sizes=(1,),
            mode=lax.GatherScatterMode.PROMISE_IN_BOUNDS)
    return v


def _issue_table(src, idxall, scals, sem, t, row0=0):
    for j in range(NCH):
        r = row0 + t * NCH + j
        pltpu.async_copy(src.at[idxall.at[r]], scals.at[r], sem)


def _drain_table(dummy, dummyv, sem):
    pltpu.make_async_copy(dummy, dummyv, sem).wait()


def _pool_step(scals, accs, t, row0, wts1=None, woff=0):
    lanes = _lanes()
    base20 = lanes * L
    out = []
    for g in range(NGRP):
        a = accs[g]
        for l in range(L):
            pv = base20 + (g * LANE * L + l)
            row = (row0 + t * NCH) + lax.shift_right_logical(pv, 7)
            col = jnp.bitwise_and(pv, 127)
            vals = plsc.load_gather(scals, [row, col])
            if wts1 is not None:
                wl = plsc.load_gather(wts1, [woff + pv])
                a = a + vals * wl
            else:
                a = a + vals
        out.append(a)
    return tuple(out)


def _dense_part(ffv, wvecv):
    lanes = _lanes()
    wv = wvecv[:]
    dense = []
    for g in range(NGRP):
        def dense_body(k, dv):
            x = _hsum(ffv[g * LANE + k, :] * wv)
            return jnp.where(lanes == k, dv + x, dv)

        dense.append(lax.fori_loop(0, LANE, dense_body,
                                   jnp.zeros((LANE,), jnp.float32)))
    return dense


def _sc_pool_body(contrib, gidxw, ffp, wvec, dummy, acc_out,
                  idxall, scals, ffv, wvecv, outv, dummyv, sem):
    wid, base = _wid_base()
    pltpu.sync_copy(gidxw.at[wid], idxall)
    pltpu.sync_copy(ffp.at[pl.ds(base, NB)], ffv)
    pltpu.sync_copy(wvec, wvecv)

    _issue_table(contrib, idxall, scals, sem, 0)
    dense = _dense_part(ffv, wvecv)

    def table_body(t, carry):
        @pl.when(t + 1 < TG)
        def _():
            _issue_table(contrib, idxall, scals, sem, t + 1)
        _drain_table(dummy, dummyv, sem)
        return _pool_step(scals, carry, t, 0)

    accs = lax.fori_loop(0, TG, table_body,
                         (dense[0], dense[1]))
    for g in range(NGRP):
        outv[pl.ds(g * LANE, LANE)] = accs[g]
    pltpu.sync_copy(outv, acc_out.at[pl.ds(base, NB)])


_WROW0 = TG * NCH


def _sc_final_body(acc_in, contrib, contrib2, gidxw, gidx2w, wtsr, out,
                   idxall, scals, wtsv, accv, outv, dummyv, sem):
    wid, base = _wid_base()
    pltpu.sync_copy(gidxw.at[wid], idxall.at[pl.ds(0, TG * NCH)])
    pltpu.sync_copy(gidx2w.at[wid], idxall.at[pl.ds(_WROW0, TW * NCH)])
    pltpu.sync_copy(wtsr.at[wid], wtsv)
    pltpu.sync_copy(acc_in.at[pl.ds(base, NB)], accv)

    _issue_table(contrib, idxall, scals, sem, 0)

    def table_body(t, carry):
        @pl.when(t + 1 < TG)
        def _():
            _issue_table(contrib, idxall, scals, sem, t + 1)
        _drain_table(contrib2, dummyv, sem)
        return _pool_step(scals, carry, t, 0)

    zero = jnp.zeros((LANE,), jnp.float32)
    accs = lax.fori_loop(0, TG, table_body, (zero, zero))

    _issue_table(contrib2, idxall, scals, sem, 0, row0=_WROW0)
    _issue_table(contrib2, idxall, scals, sem, 1, row0=_WROW0)
    for tw in range(TW):
        _drain_table(contrib2, dummyv, sem)
        accs = _pool_step(scals, accs, tw, _WROW0,
                          wts1=wtsv, woff=tw * NBL)

    for g in range(NGRP):
        x = accs[g] + accv[pl.ds(g * LANE, LANE)]
        outv[pl.ds(g * LANE, LANE)] = 1.0 / (1.0 + jnp.exp(-x))
    pltpu.sync_copy(outv, out.at[pl.ds(base, NB)])


def _sc_params():
    return dict(
        mesh=plsc.VectorSubcoreMesh(core_axis_name="c", subcore_axis_name="s"),
        compiler_params=pltpu.CompilerParams(use_tc_tiling_on_sc=False,
                                             needs_layout_passes=False),
    )


@functools.cache
def _sc_pool_kernel():
    return functools.partial(
        pl.kernel,
        out_type=jax.ShapeDtypeStruct((B,), jnp.float32),
        scratch_types=[
            pltpu.VMEM((TG * NCH, 128), jnp.int32),
            pltpu.VMEM((TG * NCH, 128), jnp.float32),
            pltpu.VMEM((NB, LANE), jnp.float32),
            pltpu.VMEM((LANE,), jnp.float32),
            pltpu.VMEM((NB,), jnp.float32),
            pltpu.VMEM((NCH, 128), jnp.float32),
            pltpu.SemaphoreType.DMA,
        ],
        **_sc_params(),
    )(_sc_pool_body)


@functools.cache
def _sc_final_kernel():
    nrow = (TG + TW) * NCH
    return functools.partial(
        pl.kernel,
        out_type=jax.ShapeDtypeStruct((B,), jnp.float32),
        scratch_types=[
            pltpu.VMEM((nrow, 128), jnp.int32),
            pltpu.VMEM((nrow, 128), jnp.float32),
            pltpu.VMEM((TW * NBL,), jnp.float32),
            pltpu.VMEM((NB,), jnp.float32),
            pltpu.VMEM((NB,), jnp.float32),
            pltpu.VMEM((NCH, 128), jnp.float32),
            pltpu.SemaphoreType.DMA,
        ],
        **_sc_params(),
    )(_sc_final_body)


def _group_idx(idx_slab, nt):
    offs = (jnp.arange(nt, dtype=jnp.int32) * VP)[:, None, None]
    g = (idx_slab.astype(jnp.int32) + offs).reshape(nt, NW, NCH, 128)
    return jnp.transpose(g, (1, 0, 2, 3)).reshape(NW, nt * NCH, 128)


def kernel(float_features, tables, weighted_tables, idscore_weights,
           Wd, bd, Wo, bo, idlist_indices, idscore_indices):
    wo_mean = jnp.mean(Wo, axis=1)
    wd_vec = Wd @ wo_mean[:8]
    const = jnp.dot(bd, wo_mean[:8]) + jnp.mean(bo)
    wvec = jnp.concatenate(
        [wd_vec, const[None], jnp.zeros((LANE - F - 1,), jnp.float32)])
    ffp = jnp.concatenate(
        [float_features,
         jnp.ones((B, 1), jnp.float32),
         jnp.zeros((B, LANE - F - 1), jnp.float32)], axis=1)
    wot = wo_mean[8:8 + T * D].reshape(T, D)
    wow = wo_mean[8 + T * D:].reshape(TW, D)

    tabT = jnp.transpose(tables, (0, 2, 1))
    wtabT = jnp.transpose(weighted_tables, (0, 2, 1))
    wot3 = wot.reshape(T, 1, D)
    wow3 = wow.reshape(TW, 1, D)
    contribA = _contract(tabT, wot3, 0, TG).reshape(TG * VP)
    contribB = _contract(tabT, wot3, 1, TG).reshape(TG * VP)
    contrib2 = _contract(wtabT, wow3, 0, TW).reshape(TW * VP)

    gidxA = _group_idx(idlist_indices[:TG], TG)
    gidxB = _group_idx(idlist_indices[TG:], TG)
    gidxW = _group_idx(idscore_indices, TW)
    wtsr = jnp.transpose(idscore_weights.reshape(TW, NW, NBL),
                         (1, 0, 2)).reshape(NW, TW * NBL)
    dummy = jnp.zeros((NCH, 128), jnp.float32)

    acc = _sc_pool_kernel()(contribA, gidxA, ffp, wvec, dummy)
    return _sc_final_kernel()(acc, contribB, contrib2, gidxB, gidxW, wtsr)


# speedup vs baseline: 1.1795x; 1.0572x over previous
import functools

import jax
import jax.numpy as jnp
from jax import lax
from jax.experimental import pallas as pl
from jax.experimental.pallas import tpu as pltpu
from jax.experimental.pallas import tpu_sc as plsc

B, L, V, D, T, TW, F = 1024, 20, 100000, 64, 26, 2, 10
NC, NS = 2, 16
NW = NC * NS
NB = B // NW
NBL = NB * L
NCH = NBL // 128
LANE = 16
VP = 102400
BV = 51200
TGA = 21
TGB = T - TGA
NGRP = NB // LANE


BH = BV // 2


def _contract_body(w_ref, tlo_ref, thi_ref, out_ref):
    w1 = w_ref[0, 0]
    out_ref[0, :BH] = jnp.sum(tlo_ref[0] * w1[:, None], axis=0)
    out_ref[0, BH:] = jnp.sum(thi_ref[0] * w1[:, None], axis=0)


def _contract(tabT, wr, base, nt):
    lasth = (V - 1) // BH

    def tab_spec(half):
        return pl.BlockSpec(
            (1, D, BH),
            lambda t, vc: (base + t, 0,
                           jnp.minimum(2 * vc + half, lasth)))

    return pl.pallas_call(
        _contract_body,
        grid=(nt, VP // BV),
        in_specs=[
            pl.BlockSpec((1, 1, D), lambda t, vc: (base + t, 0, 0)),
            tab_spec(0),
            tab_spec(1),
        ],
        out_specs=pl.BlockSpec(
            (1, BV), lambda t, vc: (0, t * (VP // BV) + vc)),
        out_shape=jax.ShapeDtypeStruct((1, nt * VP), jnp.float32),
    )(wr, tabT, tabT)


def _wid_base():
    c = lax.axis_index("c")
    s = lax.axis_index("s")
    wid = s * NC + c
    return wid, wid * NB


def _lanes():
    return lax.iota(jnp.int32, LANE)


def _hsum(v):
    lanes = _lanes()
    dnums = lax.GatherDimensionNumbers(
        offset_dims=(), collapsed_slice_dims=(0,), start_index_map=(0,))
    for sh in (8, 4, 2, 1):
        perm = jnp.bitwise_xor(lanes, sh)
        v = v + lax.gather(
            v, perm[:, None], dnums, slice_sizes=(1,),
            mode=lax.GatherScatterMode.PROMISE_IN_BOUNDS)
    return v


def _issue_table(src, idxall, scals, sem, t, dstrow, row0=0):
    for j in range(NCH):
        r = row0 + t * NCH + j
        pltpu.async_copy(src.at[idxall.at[r]], scals.at[dstrow + j], sem)


def _drain_table(dummy, dummyv, sem):
    pltpu.make_async_copy(dummy, dummyv, sem).wait()


def _pool_step(scals, accs, dstrow, wts1=None, woff=0):
    lanes = _lanes()
    base20 = lanes * L
    out = []
    for g in range(NGRP):
        a = accs[g]
        for l in range(L):
            pv = base20 + (g * LANE * L + l)
            row = dstrow + lax.shift_right_logical(pv, 7)
            col = jnp.bitwise_and(pv, 127)
            vals = plsc.load_gather(scals, [row, col])
            if wts1 is not None:
                wl = plsc.load_gather(wts1, [woff + pv])
                a = a + vals * wl
            else:
                a = a + vals
        out.append(a)
    return tuple(out)


def _dense_part(ffv, wvecv):
    lanes = _lanes()
    wv = wvecv[:]
    dense = []
    for g in range(NGRP):
        def dense_body(k, dv):
            x = _hsum(ffv[g * LANE + k, :] * wv)
            return jnp.where(lanes == k, dv + x, dv)

        dense.append(lax.fori_loop(0, LANE, dense_body,
                                   jnp.zeros((LANE,), jnp.float32)))
    return dense


def _ring_pool_loop(nt, contrib, idxall, scals, dummy, dummyv, sem, init):
    _issue_table(contrib, idxall, scals, sem, 0, 0)

    def table_body(t, carry):
        slot = jnp.bitwise_and(t, 1) * NCH
        nslot = jnp.bitwise_and(t + 1, 1) * NCH

        @pl.when(t + 1 < nt)
        def _():
            _issue_table(contrib, idxall, scals, sem, t + 1, nslot)
        _drain_table(dummy, dummyv, sem)
        return _pool_step(scals, carry, slot)

    return lax.fori_loop(0, nt, table_body, init)


def _sc_pool_body(nt, contrib, gidxw, ffp, wvec, dummy, acc_out,
                  idxall, scals, ffv, wvecv, outv, dummyv, sem):
    wid, base = _wid_base()
    pltpu.sync_copy(gidxw.at[wid], idxall)
    pltpu.sync_copy(ffp.at[pl.ds(base, NB)], ffv)
    pltpu.sync_copy(wvec, wvecv)

    dense = _dense_part(ffv, wvecv)
    accs = _ring_pool_loop(nt, contrib, idxall, scals, dummy, dummyv, sem,
                           (dense[0], dense[1]))
    for g in range(NGRP):
        outv[pl.ds(g * LANE, LANE)] = accs[g]
    pltpu.sync_copy(outv, acc_out.at[pl.ds(base, NB)])


def _sc_mid_body(nt, acc_in, contrib, gidxw, dummy, acc_out,
                 idxall, scals, accv, outv, dummyv, sem):
    wid, base = _wid_base()
    pltpu.sync_copy(gidxw.at[wid], idxall)
    pltpu.sync_copy(acc_in.at[pl.ds(base, NB)], accv)

    zero = jnp.zeros((LANE,), jnp.float32)
    accs = _ring_pool_loop(nt, contrib, idxall, scals, dummy, dummyv, sem,
                           (zero, zero))
    for g in range(NGRP):
        outv[pl.ds(g * LANE, LANE)] = accs[g] + accv[pl.ds(g * LANE, LANE)]
    pltpu.sync_copy(outv, acc_out.at[pl.ds(base, NB)])


def _sc_final_body(acc_in, contrib2, gidx2w, wtsr, dummy, out,
                   idxall, scals, wtsv, accv, outv, dummyv, sem):
    wid, base = _wid_base()
    pltpu.sync_copy(gidx2w.at[wid], idxall)
    pltpu.sync_copy(wtsr.at[wid], wtsv)
    pltpu.sync_copy(acc_in.at[pl.ds(base, NB)], accv)

    _issue_table(contrib2, idxall, scals, sem, 0, 0)
    _issue_table(contrib2, idxall, scals, sem, 1, NCH)
    zero = jnp.zeros((LANE,), jnp.float32)
    accs = (zero, zero)
    for tw in range(TW):
        _drain_table(dummy, dummyv, sem)
        accs = _pool_step(scals, accs, tw * NCH, wts1=wtsv, woff=tw * NBL)

    for g in range(NGRP):
        x = accs[g] + accv[pl.ds(g * LANE, LANE)]
        outv[pl.ds(g * LANE, LANE)] = 1.0 / (1.0 + jnp.exp(-x))
    pltpu.sync_copy(outv, out.at[pl.ds(base, NB)])


def _sc_params():
    return dict(
        mesh=plsc.VectorSubcoreMesh(core_axis_name="c", subcore_axis_name="s"),
        compiler_params=pltpu.CompilerParams(use_tc_tiling_on_sc=False,
                                             needs_layout_passes=False),
    )


@functools.cache
def _sc_pool_kernel(nt):
    return functools.partial(
        pl.kernel,
        out_type=jax.ShapeDtypeStruct((B,), jnp.float32),
        scratch_types=[
            pltpu.VMEM((nt * NCH, 128), jnp.int32),
            pltpu.VMEM((2 * NCH, 128), jnp.float32),
            pltpu.VMEM((NB, LANE), jnp.float32),
            pltpu.VMEM((LANE,), jnp.float32),
            pltpu.VMEM((NB,), jnp.float32),
            pltpu.VMEM((NCH, 128), jnp.float32),
            pltpu.SemaphoreType.DMA,
        ],
        **_sc_params(),
    )(functools.partial(_sc_pool_body, nt))


@functools.cache
def _sc_mid_kernel(nt):
    return functools.partial(
        pl.kernel,
        out_type=jax.ShapeDtypeStruct((B,), jnp.float32),
        scratch_types=[
            pltpu.VMEM((nt * NCH, 128), jnp.int32),
            pltpu.VMEM((2 * NCH, 128), jnp.float32),
            pltpu.VMEM((NB,), jnp.float32),
            pltpu.VMEM((NB,), jnp.float32),
            pltpu.VMEM((NCH, 128), jnp.float32),
            pltpu.SemaphoreType.DMA,
        ],
        **_sc_params(),
    )(functools.partial(_sc_mid_body, nt))


@functools.cache
def _sc_final_kernel():
    return functools.partial(
        pl.kernel,
        out_type=jax.ShapeDtypeStruct((B,), jnp.float32),
        scratch_types=[
            pltpu.VMEM((TW * NCH, 128), jnp.int32),
            pltpu.VMEM((TW * NCH, 128), jnp.float32),
            pltpu.VMEM((TW * NBL,), jnp.float32),
            pltpu.VMEM((NB,), jnp.float32),
            pltpu.VMEM((NB,), jnp.float32),
            pltpu.VMEM((NCH, 128), jnp.float32),
            pltpu.SemaphoreType.DMA,
        ],
        **_sc_params(),
    )(_sc_final_body)


def _group_idx(idx_slab, nt):
    offs = (jnp.arange(nt, dtype=jnp.int32) * VP)[:, None, None]
    g = (idx_slab.astype(jnp.int32) + offs).reshape(nt, NW, NCH, 128)
    return jnp.transpose(g, (1, 0, 2, 3)).reshape(NW, nt * NCH, 128)


def kernel(float_features, tables, weighted_tables, idscore_weights,
           Wd, bd, Wo, bo, idlist_indices, idscore_indices):
    wo_mean = jnp.mean(Wo, axis=1)
    wd_vec = Wd @ wo_mean[:8]
    const = jnp.dot(bd, wo_mean[:8]) + jnp.mean(bo)
    wvec = jnp.concatenate(
        [wd_vec, const[None], jnp.zeros((LANE - F - 1,), jnp.float32)])
    ffp = jnp.concatenate(
        [float_features,
         jnp.ones((B, 1), jnp.float32),
         jnp.zeros((B, LANE - F - 1), jnp.float32)], axis=1)
    wot = wo_mean[8:8 + T * D].reshape(T, D)
    wow = wo_mean[8 + T * D:].reshape(TW, D)

    tabT = jnp.transpose(tables, (0, 2, 1))
    wtabT = jnp.transpose(weighted_tables, (0, 2, 1))
    wot3 = wot.reshape(T, 1, D)
    wow3 = wow.reshape(TW, 1, D)
    contribA = _contract(tabT, wot3, 0, TGA).reshape(TGA * VP)
    contribB = _contract(tabT, wot3, TGA, TGB).reshape(TGB * VP)
    contrib2 = _contract(wtabT, wow3, 0, TW).reshape(TW * VP)

    gidxA = _group_idx(idlist_indices[:TGA], TGA)
    gidxB = _group_idx(idlist_indices[TGA:], TGB)
    gidxW = _group_idx(idscore_indices, TW)
    wtsr = jnp.transpose(idscore_weights.reshape(TW, NW, NBL),
                         (1, 0, 2)).reshape(NW, TW * NBL)
    dummy = jnp.zeros((NCH, 128), jnp.float32)

    acc = _sc_pool_kernel(TGA)(contribA, gidxA, ffp, wvec, dummy)
    acc2 = _sc_mid_kernel(TGB)(acc, contribB, gidxB, dummy)
    return _sc_final_kernel()(acc2, contrib2, gidxW, wtsr, dummy)
